# baseline (speedup 1.0000x reference)
import functools

import jax
import jax.numpy as jnp
from jax import lax
from jax.experimental import pallas as pl
from jax.experimental.pallas import tpu as pltpu
from jax.experimental.pallas import tpu_sc as plsc

B, N, EMB = 1024, 32, 64
NUM_BUCKETS = 201
TVOL = NUM_BUCKETS * EMB

NC, NS = 2, 16
NW = NC * NS
HROW = N // 2
HCHUNK = B // 2


def _body(dist_hbm, table_hbm, out_hbm,
          dv0, dv1, buf0, buf1, table_v,
          dist_sem, store_sem0, store_sem1):
    w = lax.axis_index("s") * NC + lax.axis_index("c")
    dv = (dv0, dv1)
    buf = (buf0, buf1)
    store_sem = (store_sem0, store_sem1)

    pltpu.sync_copy(table_hbm, table_v)

    def fire_dist(h2):
        pltpu.async_copy(
            dist_hbm.at[w, pl.ds(h2 * HROW, HROW), :], dv[h2], dist_sem)

    def wait_dist(h2):
        pltpu.make_async_copy(
            dist_hbm.at[0, pl.ds(0, HROW), :], dv[h2], dist_sem).wait()

    def drain_store(h):
        pltpu.make_async_copy(
            buf[h], out_hbm.at[0, :, pl.ds(0, HCHUNK)], store_sem[h]).wait()

    def gather_chunk(h2, j16, h):
        def c_body(c, carry):
            v = dv[h2][j16, pl.ds(h * HCHUNK + c * 16, 16)]
            idx = jnp.clip(v, 0.0, float(NUM_BUCKETS - 1)).astype(jnp.int32)
            for e0 in range(0, EMB, 16):
                gs = [plsc.load_gather(table_v,
                                       [idx + ((e0 + j) * NUM_BUCKETS)])
                      for j in range(16)]
                for j in range(16):
                    buf[h][e0 + j, pl.ds(c * 16, 16)] = gs[j]
            return carry

        lax.fori_loop(0, HCHUNK // 16, c_body, 0)

    fire_dist(0)

    for h2 in range(2):
        wait_dist(h2)
        if h2 == 0:
            fire_dist(1)

        def j_body(j16, carry, h2=h2):
            blk = w * N + h2 * HROW + j16

            for h in range(2):
                if h2 == 0:
                    @pl.when(j16 >= 1)
                    def _drain():
                        drain_store(h)
                else:
                    drain_store(h)

                gather_chunk(h2, j16, h)
                pltpu.async_copy(
                    buf[h], out_hbm.at[blk, :, pl.ds(h * HCHUNK, HCHUNK)],
                    store_sem[h])
            return carry

        lax.fori_loop(0, HROW, j_body, 0)

    drain_store(0)
    drain_store(1)


def kernel(distance_matrix, table):
    dist_t = distance_matrix.transpose(1, 2, 0)
    table_flat = table.T.reshape(TVOL)
    mesh = plsc.VectorSubcoreMesh(core_axis_name="c", subcore_axis_name="s")
    k = functools.partial(
        pl.kernel,
        out_type=jax.ShapeDtypeStruct((N * N, EMB, B), jnp.float32),
        mesh=mesh,
        scratch_types=[
            pltpu.VMEM((HROW, B), jnp.float32),
            pltpu.VMEM((HROW, B), jnp.float32),
            pltpu.VMEM((EMB, HCHUNK), jnp.float32),
            pltpu.VMEM((EMB, HCHUNK), jnp.float32),
            pltpu.VMEM((TVOL,), jnp.float32),
            pltpu.SemaphoreType.DMA,
            pltpu.SemaphoreType.DMA,
            pltpu.SemaphoreType.DMA,
        ],
        compiler_params=pltpu.CompilerParams(
            use_tc_tiling_on_sc=True, needs_layout_passes=False),
    )(_body)
    out_t = k(dist_t, table_flat)
    return out_t.reshape(N, N, EMB, B).transpose(3, 0, 1, 2)
